# Initial kernel scaffold; baseline (speedup 1.0000x reference)
import jax
import jax.numpy as jnp
from jax.experimental import pallas as pl


def kernel(x, params):
    raise NotImplementedError("write your pallas kernel here")


# speedup vs baseline: 3.2137x; 3.2137x over previous
import functools

import jax
import jax.numpy as jnp
from jax import lax
from jax.experimental import pallas as pl
from jax.experimental.pallas import tpu as pltpu

F32 = jnp.float32
HI = lax.Precision.DEFAULT

NBK = 2
KNN = 16
NV = 2048
NT = NBK * NV

_INTERPRET = False


def _pc(body, **kw):
    return pl.pallas_call(body, interpret=_INTERPRET, **kw)


def _ft_body(x_ref, w_ref, b_ref, o_ref):
    acc = jnp.dot(x_ref[...], w_ref[...], preferred_element_type=F32,
                  precision=HI)
    o_ref[...] = jnp.maximum(acc + b_ref[...], 0.0)


def _feature_transform(x, ft_W, ft_b):
    n, in_feat = x.shape
    tc = ft_W[0].shape[1]
    w6 = jnp.zeros((in_feat, NBK * tc), F32)
    w6 = w6.at[0:3, 0:tc].set(ft_W[0])
    w6 = w6.at[3:6, tc:2 * tc].set(ft_W[1])
    b6 = jnp.concatenate([ft_b[0], ft_b[1]])[None, :]
    out = _pc(_ft_body,
              out_shape=jax.ShapeDtypeStruct((n, NBK * tc), F32))(x, w6, b6)
    return jnp.concatenate([out[:, :tc], out[:, tc:]], axis=0)


def _pre_body(hi_ref, hT_ref, wl_ref, wr_ref, gl_ref, gr_ref, nbr_ref, *, R):
    b = pl.program_id(0)
    r = pl.program_id(1)
    hi = hi_ref[...]
    gl_ref[...] = jnp.dot(hi, wl_ref[...], preferred_element_type=F32,
                          precision=HI)
    gr_ref[...] = jnp.dot(hi, wr_ref[...], preferred_element_type=F32,
                          precision=HI)
    hT = hT_ref[0]
    sqj = jnp.sum(hT * hT, axis=0)
    sqi = jnp.sum(hi * hi, axis=1)
    cross = jnp.dot(hi, hT, preferred_element_type=F32, precision=HI)
    d = sqi[:, None] + sqj[None, :] - 2.0 * cross
    rowid = lax.broadcasted_iota(jnp.int32, (R, NV), 0) + r * R
    colid = lax.broadcasted_iota(jnp.int32, (R, NV), 1)
    d = jnp.where(colid == rowid, d + 1e10, d)
    cols = []
    for _ in range(KNN):
        m = jnp.min(d, axis=1, keepdims=True)
        idx = jnp.min(jnp.where(d <= m, colid, NV), axis=1)
        cols.append(idx + b * NV)
        d = jnp.where(colid == idx[:, None], jnp.float32(1e30), d)
    nbr_ref[...] = jnp.stack(cols, axis=1)


def _proj_and_knn(h, wl, wr, R=256):
    din = h.shape[1]
    dout = wl.shape[1]
    hT = h.reshape(NBK, NV, din).transpose(0, 2, 1)
    nb = NV // R
    grid = (NBK, nb)
    gl, gr, nbr = _pc(
        functools.partial(_pre_body, R=R),
        grid=grid,
        in_specs=[
            pl.BlockSpec((R, din), lambda b, r: (b * (NV // R) + r, 0)),
            pl.BlockSpec((1, din, NV), lambda b, r: (b, 0, 0)),
            pl.BlockSpec((din, dout), lambda b, r: (0, 0)),
            pl.BlockSpec((din, dout), lambda b, r: (0, 0)),
        ],
        out_specs=[
            pl.BlockSpec((R, dout), lambda b, r: (b * (NV // R) + r, 0)),
            pl.BlockSpec((R, dout), lambda b, r: (b * (NV // R) + r, 0)),
            pl.BlockSpec((R, KNN), lambda b, r: (b * (NV // R) + r, 0)),
        ],
        out_shape=[
            jax.ShapeDtypeStruct((NT, dout), F32),
            jax.ShapeDtypeStruct((NT, dout), F32),
            jax.ShapeDtypeStruct((NT, KNN), jnp.int32),
        ],
    )(h, hT, wl, wr)
    return gl, gr, nbr


def _att_body(gl_ref, gj_ref, att_ref, bias_ref, o_ref, *, Bn, dout):
    gl = gl_ref[...]
    gj = gj_ref[...].reshape(Bn, KNN, dout)
    s = gl[:, None, :] + gj
    lr = jnp.where(s > 0, s, 0.2 * s)
    e = jnp.sum(lr * att_ref[...][None, :, :], axis=2)
    m = jnp.max(e, axis=1, keepdims=True)
    ex = jnp.exp(e - m)
    alpha = ex / jnp.sum(ex, axis=1, keepdims=True)
    out = jnp.sum(alpha[:, :, None] * gj, axis=1) + bias_ref[...]
    o_ref[...] = jnp.maximum(out, 0.0)


def _attention(gl, gj_flat, att, bias, Bn=128):
    dout = gl.shape[1]
    grid = (NT // Bn,)
    return _pc(
        functools.partial(_att_body, Bn=Bn, dout=dout),
        grid=grid,
        in_specs=[
            pl.BlockSpec((Bn, dout), lambda i: (i, 0)),
            pl.BlockSpec((Bn * KNN, dout), lambda i: (i, 0)),
            pl.BlockSpec((1, dout), lambda i: (0, 0)),
            pl.BlockSpec((1, dout), lambda i: (0, 0)),
        ],
        out_specs=pl.BlockSpec((Bn, dout), lambda i: (i, 0)),
        out_shape=jax.ShapeDtypeStruct((NT, dout), F32),
    )(gl, gj_flat, att[None, :], bias[None, :])


def _gather_rows(gr, nbr):
    return jnp.take(gr, nbr.reshape(-1), axis=0)


def _mlp_body(z_ref, w1_ref, b1_ref, w2_ref, b2_ref, w3_ref, b3_ref, o_ref):
    h = jnp.dot(z_ref[...], w1_ref[...], preferred_element_type=F32,
                precision=HI) + b1_ref[...]
    h = jnp.maximum(h, 0.0)
    h = jnp.dot(h, w2_ref[...], preferred_element_type=F32,
                precision=HI) + b2_ref[...]
    h = jnp.maximum(h, 0.0)
    o_ref[...] = jnp.dot(h, w3_ref[...], preferred_element_type=F32,
                         precision=HI) + b3_ref[...]


def _mlp(z, mlp_params):
    (w1, b1), (w2, b2), (w3, b3) = mlp_params
    return _pc(_mlp_body,
               out_shape=jax.ShapeDtypeStruct((z.shape[0], w3.shape[1]), F32),
               )(z, w1, b1[None, :], w2, b2[None, :], w3, b3[None, :])


def kernel(x, params):
    h = _feature_transform(x, params['ft_W'], params['ft_b'])
    out_list = [h]
    for l in range(4):
        lp = params['gat'][l]
        net_in = out_list[0] if l == 0 else jnp.concatenate(out_list[-2:],
                                                            axis=1)
        gl, gr, nbr = _proj_and_knn(net_in, lp['Wl'], lp['Wr'])
        gj = _gather_rows(gr, nbr)
        out_list.append(_attention(gl, gj, lp['att'], lp['bias']))
    last = out_list[-1]
    n = NV
    z = jnp.concatenate([last[:n], last[n:]], axis=1)
    return _mlp(z, params['mlp'])


# speedup vs baseline: 5.1493x; 1.6023x over previous
import functools

import jax
import jax.numpy as jnp
from jax import lax
from jax.experimental import pallas as pl
from jax.experimental.pallas import tpu as pltpu
from jax.experimental.pallas import tpu_sc as plsc

F32 = jnp.float32
HI = lax.Precision.DEFAULT

NBK = 2
KNN = 16
NV = 2048
NT = NBK * NV

_INTERPRET = False


def _pc(body, **kw):
    return pl.pallas_call(body, interpret=_INTERPRET, **kw)


def _ft_body(x_ref, w_ref, b_ref, o_ref):
    acc = jnp.dot(x_ref[...], w_ref[...], preferred_element_type=F32,
                  precision=HI)
    o_ref[...] = jnp.maximum(acc + b_ref[...], 0.0)


def _feature_transform(x, ft_W, ft_b):
    n, in_feat = x.shape
    tc = ft_W[0].shape[1]
    w6 = jnp.zeros((in_feat, NBK * tc), F32)
    w6 = w6.at[0:3, 0:tc].set(ft_W[0])
    w6 = w6.at[3:6, tc:2 * tc].set(ft_W[1])
    b6 = jnp.concatenate([ft_b[0], ft_b[1]])[None, :]
    out = _pc(_ft_body,
              out_shape=jax.ShapeDtypeStruct((n, NBK * tc), F32))(x, w6, b6)
    return jnp.concatenate([out[:, :tc], out[:, tc:]], axis=0)


def _pre_body(hi_ref, hT_ref, wl_ref, wr_ref, gl_ref, gr_ref, nbr_ref, *, R):
    b = pl.program_id(0)
    r = pl.program_id(1)
    hi = hi_ref[...]
    gl_ref[...] = jnp.dot(hi, wl_ref[...], preferred_element_type=F32,
                          precision=HI)
    gr_ref[...] = jnp.dot(hi, wr_ref[...], preferred_element_type=F32,
                          precision=HI)
    hT = hT_ref[0]
    sqj = jnp.sum(hT * hT, axis=0)
    sqi = jnp.sum(hi * hi, axis=1)
    cross = jnp.dot(hi, hT, preferred_element_type=F32, precision=HI)
    d = sqi[:, None] + sqj[None, :] - 2.0 * cross
    rowid = lax.broadcasted_iota(jnp.int32, (R, NV), 0) + r * R
    colid = lax.broadcasted_iota(jnp.int32, (R, NV), 1)
    d = jnp.where(colid == rowid, d + 1e10, d)
    cols = []
    for _ in range(KNN):
        m = jnp.min(d, axis=1, keepdims=True)
        idx = jnp.min(jnp.where(d <= m, colid, NV), axis=1)
        cols.append(idx + b * NV)
        d = jnp.where(colid == idx[:, None], jnp.float32(1e30), d)
    nbr_ref[...] = jnp.stack(cols, axis=1)


def _proj_and_knn(h, wl, wr, R=256):
    din = h.shape[1]
    dout = wl.shape[1]
    hT = h.reshape(NBK, NV, din).transpose(0, 2, 1)
    nb = NV // R
    grid = (NBK, nb)
    gl, gr, nbr = _pc(
        functools.partial(_pre_body, R=R),
        grid=grid,
        in_specs=[
            pl.BlockSpec((R, din), lambda b, r: (b * (NV // R) + r, 0)),
            pl.BlockSpec((1, din, NV), lambda b, r: (b, 0, 0)),
            pl.BlockSpec((din, dout), lambda b, r: (0, 0)),
            pl.BlockSpec((din, dout), lambda b, r: (0, 0)),
        ],
        out_specs=[
            pl.BlockSpec((R, dout), lambda b, r: (b * (NV // R) + r, 0)),
            pl.BlockSpec((R, dout), lambda b, r: (b * (NV // R) + r, 0)),
            pl.BlockSpec((R, KNN), lambda b, r: (b * (NV // R) + r, 0)),
        ],
        out_shape=[
            jax.ShapeDtypeStruct((NT, dout), F32),
            jax.ShapeDtypeStruct((NT, dout), F32),
            jax.ShapeDtypeStruct((NT, KNN), jnp.int32),
        ],
    )(h, hT, wl, wr)
    return gl, gr, nbr


def _att_body(gl_ref, gj_ref, att_ref, bias_ref, o_ref, *, Bn, dout):
    gl = gl_ref[...]
    gj = gj_ref[...].reshape(Bn, KNN, dout)
    s = gl[:, None, :] + gj
    lr = jnp.where(s > 0, s, 0.2 * s)
    e = jnp.sum(lr * att_ref[...][None, :, :], axis=2)
    m = jnp.max(e, axis=1, keepdims=True)
    ex = jnp.exp(e - m)
    alpha = ex / jnp.sum(ex, axis=1, keepdims=True)
    out = jnp.sum(alpha[:, :, None] * gj, axis=1) + bias_ref[...]
    o_ref[...] = jnp.maximum(out, 0.0)


def _attention(gl, gj_flat, att, bias, Bn=128):
    dout = gl.shape[1]
    grid = (NT // Bn,)
    return _pc(
        functools.partial(_att_body, Bn=Bn, dout=dout),
        grid=grid,
        in_specs=[
            pl.BlockSpec((Bn, dout), lambda i: (i, 0)),
            pl.BlockSpec((Bn * KNN, dout), lambda i: (i, 0)),
            pl.BlockSpec((1, dout), lambda i: (0, 0)),
            pl.BlockSpec((1, dout), lambda i: (0, 0)),
        ],
        out_specs=pl.BlockSpec((Bn, dout), lambda i: (i, 0)),
        out_shape=jax.ShapeDtypeStruct((NT, dout), F32),
    )(gl, gj_flat, att[None, :], bias[None, :])


def _gather_rows(table, nbr):
    idx = nbr.reshape(-1)
    B = idx.shape[0]
    D = table.shape[1]
    NW = 32
    bpw = B // NW
    C = max(8, min(128, (128 * 1024 // 4) // D))
    nchunk = bpw // C
    mesh = plsc.VectorSubcoreMesh(core_axis_name="c", subcore_axis_name="s")

    @functools.partial(
        pl.kernel, mesh=mesh,
        out_type=jax.ShapeDtypeStruct((B, D), F32),
        scratch_types=[
            pltpu.VMEM((C,), jnp.int32),
            pltpu.VMEM((C, D), F32),
            pltpu.SemaphoreType.DMA,
        ],
    )
    def k(table_hbm, idx_hbm, out_hbm, idx_v, rows_v, sem):
        wid = lax.axis_index("s") * 2 + lax.axis_index("c")
        base = wid * bpw

        def body(g, carry):
            off = base + g * C
            pltpu.sync_copy(idx_hbm.at[pl.ds(off, C)], idx_v)
            pltpu.async_copy(table_hbm.at[idx_v], rows_v, sem).wait()
            pltpu.sync_copy(rows_v, out_hbm.at[pl.ds(off, C)])
            return carry

        lax.fori_loop(0, nchunk, body, 0)

    return k(table, idx)


def _mlp_body(z_ref, w1_ref, b1_ref, w2_ref, b2_ref, w3_ref, b3_ref, o_ref):
    h = jnp.dot(z_ref[...], w1_ref[...], preferred_element_type=F32,
                precision=HI) + b1_ref[...]
    h = jnp.maximum(h, 0.0)
    h = jnp.dot(h, w2_ref[...], preferred_element_type=F32,
                precision=HI) + b2_ref[...]
    h = jnp.maximum(h, 0.0)
    o_ref[...] = jnp.dot(h, w3_ref[...], preferred_element_type=F32,
                         precision=HI) + b3_ref[...]


def _mlp(z, mlp_params):
    (w1, b1), (w2, b2), (w3, b3) = mlp_params
    return _pc(_mlp_body,
               out_shape=jax.ShapeDtypeStruct((z.shape[0], w3.shape[1]), F32),
               )(z, w1, b1[None, :], w2, b2[None, :], w3, b3[None, :])


def kernel(x, params):
    h = _feature_transform(x, params['ft_W'], params['ft_b'])
    out_list = [h]
    for l in range(4):
        lp = params['gat'][l]
        net_in = out_list[0] if l == 0 else jnp.concatenate(out_list[-2:],
                                                            axis=1)
        gl, gr, nbr = _proj_and_knn(net_in, lp['Wl'], lp['Wr'])
        gj = _gather_rows(gr, nbr)
        out_list.append(_attention(gl, gj, lp['att'], lp['bias']))
    last = out_list[-1]
    n = NV
    z = jnp.concatenate([last[:n], last[n:]], axis=1)
    return _mlp(z, params['mlp'])


# speedup vs baseline: 6.7860x; 1.3178x over previous
import functools

import jax
import jax.numpy as jnp
from jax import lax
from jax.experimental import pallas as pl
from jax.experimental.pallas import tpu as pltpu
from jax.experimental.pallas import tpu_sc as plsc

F32 = jnp.float32
HI = lax.Precision.DEFAULT

NBK = 2
KNN = 16
NV = 2048

_INTERPRET = False


def _pc(body, **kw):
    return pl.pallas_call(body, interpret=_INTERPRET, **kw)


def _ft_body(x_ref, w_ref, b_ref, o_ref):
    acc = jnp.dot(x_ref[...], w_ref[...], preferred_element_type=F32,
                  precision=HI)
    o_ref[...] = jnp.maximum(acc + b_ref[...], 0.0)


def _feature_transform(x, ft_W, ft_b):
    n, in_feat = x.shape
    tc = ft_W[0].shape[1]
    w6 = jnp.zeros((in_feat, NBK * tc), F32)
    w6 = w6.at[0:3, 0:tc].set(ft_W[0])
    w6 = w6.at[3:6, tc:2 * tc].set(ft_W[1])
    b6 = jnp.concatenate([ft_b[0], ft_b[1]])[None, :]
    out = _pc(_ft_body,
              out_shape=jax.ShapeDtypeStruct((n, NBK * tc), F32))(x, w6, b6)
    return out[:, :tc], out[:, tc:]


def _pre_body(hi_ref, hT_ref, wl_ref, wr_ref, gl_ref, gr_ref, nbr_ref, *, R):
    r = pl.program_id(0)
    hi = hi_ref[...]
    gl_ref[...] = jnp.dot(hi, wl_ref[...], preferred_element_type=F32,
                          precision=HI)
    gr_ref[...] = jnp.dot(hi, wr_ref[...], preferred_element_type=F32,
                          precision=HI)
    hT = hT_ref[...]
    sqj = jnp.sum(hT * hT, axis=0)
    sqi = jnp.sum(hi * hi, axis=1)
    cross = jnp.dot(hi, hT, preferred_element_type=F32, precision=HI)
    d = sqi[:, None] + sqj[None, :] - 2.0 * cross
    rowid = lax.broadcasted_iota(jnp.int32, (R, NV), 0) + r * R
    colid = lax.broadcasted_iota(jnp.int32, (R, NV), 1)
    d = jnp.where(colid == rowid, jnp.float32(1e10), jnp.maximum(d, 0.0))
    key = (lax.bitcast_convert_type(d, jnp.int32) & ~jnp.int32(NV - 1)) | colid
    cols = []
    for _ in range(KNN):
        m = jnp.min(key, axis=1, keepdims=True)
        cols.append(m[:, 0] & jnp.int32(NV - 1))
        key = jnp.where(key == m, jnp.int32(0x7FFFFFFF), key)
    nbr_ref[...] = jnp.stack(cols, axis=1)


def _proj_and_knn(h, wl, wr, R=256):
    din = h.shape[1]
    dout = wl.shape[1]
    hT = h.T
    grid = (NV // R,)
    gl, gr, nbr = _pc(
        functools.partial(_pre_body, R=R),
        grid=grid,
        in_specs=[
            pl.BlockSpec((R, din), lambda r: (r, 0)),
            pl.BlockSpec((din, NV), lambda r: (0, 0)),
            pl.BlockSpec((din, dout), lambda r: (0, 0)),
            pl.BlockSpec((din, dout), lambda r: (0, 0)),
        ],
        out_specs=[
            pl.BlockSpec((R, dout), lambda r: (r, 0)),
            pl.BlockSpec((R, dout), lambda r: (r, 0)),
            pl.BlockSpec((R, KNN), lambda r: (r, 0)),
        ],
        out_shape=[
            jax.ShapeDtypeStruct((NV, dout), F32),
            jax.ShapeDtypeStruct((NV, dout), F32),
            jax.ShapeDtypeStruct((NV, KNN), jnp.int32),
        ],
    )(h, hT, wl, wr)
    return gl, gr, nbr


def _att_body(gl_ref, gj_ref, attc_ref, bias_ref, o_ref, *, Bn, dout):
    gj = gj_ref[...]
    gl = gl_ref[...]
    glx = jnp.broadcast_to(gl[:, None, :], (Bn, KNN, dout)).reshape(
        Bn * KNN, dout)
    s = glx + gj
    lr = jnp.where(s > 0, s, 0.2 * s)
    e = jnp.dot(lr, attc_ref[...], preferred_element_type=F32,
                precision=HI)[:, 0]
    e1 = e.reshape(Bn, KNN)
    m = jnp.max(e1, axis=1, keepdims=True)
    ex = jnp.exp(e1 - m)
    alpha = ex / jnp.sum(ex, axis=1, keepdims=True)
    gj3 = gj.reshape(Bn, KNN, dout)
    out = jnp.sum(alpha[:, :, None] * gj3, axis=1) + bias_ref[...]
    o_ref[...] = jnp.maximum(out, 0.0)


def _attention(gl, gj_flat, att, bias, Bn=128):
    dout = gl.shape[1]
    rows = gl.shape[0]
    grid = (rows // Bn,)
    attc = jnp.broadcast_to(att[:, None], (dout, 128))
    return _pc(
        functools.partial(_att_body, Bn=Bn, dout=dout),
        grid=grid,
        in_specs=[
            pl.BlockSpec((Bn, dout), lambda i: (i, 0)),
            pl.BlockSpec((Bn * KNN, dout), lambda i: (i, 0)),
            pl.BlockSpec((dout, 128), lambda i: (0, 0)),
            pl.BlockSpec((1, dout), lambda i: (0, 0)),
        ],
        out_specs=pl.BlockSpec((Bn, dout), lambda i: (i, 0)),
        out_shape=jax.ShapeDtypeStruct((rows, dout), F32),
    )(gl, gj_flat, attc, bias[None, :])


def _gather_rows(table, idx):
    B = idx.shape[0]
    D = table.shape[1]
    NW = 32
    bpw = B // NW
    C = 64 if D >= 512 else 128
    nchunk = bpw // C
    mesh = plsc.VectorSubcoreMesh(core_axis_name="c", subcore_axis_name="s")

    @functools.partial(
        pl.kernel, mesh=mesh,
        out_type=jax.ShapeDtypeStruct((B, D), F32),
        scratch_types=[
            pltpu.VMEM((bpw,), jnp.int32),
            pltpu.VMEM((C, D), F32),
            pltpu.VMEM((C, D), F32),
            pltpu.SemaphoreType.DMA,
            pltpu.SemaphoreType.DMA,
        ],
    )
    def k(table_hbm, idx_hbm, out_hbm, idx_v, rows0_v, rows1_v, sem0, sem1):
        wid = lax.axis_index("s") * 2 + lax.axis_index("c")
        base = wid * bpw
        pltpu.sync_copy(idx_hbm.at[pl.ds(base, bpw)], idx_v)
        bufs = (rows0_v, rows1_v)
        sems = (sem0, sem1)
        pltpu.async_copy(table_hbm.at[idx_v.at[pl.ds(0, C)]], rows0_v, sem0)

        def body(p, carry):
            for bsel in range(2):
                g = p + bsel
                pltpu.make_async_copy(table_hbm.at[idx_v.at[pl.ds(0, C)]],
                                      bufs[bsel], sems[bsel]).wait()

                @pl.when(g + 1 < nchunk)
                def _():
                    pltpu.async_copy(
                        table_hbm.at[idx_v.at[pl.ds((g + 1) * C, C)]],
                        bufs[1 - bsel], sems[1 - bsel])

                pltpu.sync_copy(bufs[bsel],
                                out_hbm.at[pl.ds(base + g * C, C)])
            return carry

        lax.fori_loop(0, nchunk // 2, lambda p, c: body(p * 2, c), 0,
                      unroll=False)

    return k(table, idx)


def _mlp_body(z0_ref, z1_ref, w1a_ref, w1b_ref, b1_ref, w2_ref, b2_ref,
              w3_ref, b3_ref, o_ref):
    h = (jnp.dot(z0_ref[...], w1a_ref[...], preferred_element_type=F32,
                 precision=HI)
         + jnp.dot(z1_ref[...], w1b_ref[...], preferred_element_type=F32,
                   precision=HI) + b1_ref[...])
    h = jnp.maximum(h, 0.0)
    h = jnp.dot(h, w2_ref[...], preferred_element_type=F32,
                precision=HI) + b2_ref[...]
    h = jnp.maximum(h, 0.0)
    o_ref[...] = jnp.dot(h, w3_ref[...], preferred_element_type=F32,
                         precision=HI) + b3_ref[...]


def _mlp(z0, z1, mlp_params):
    (w1, b1), (w2, b2), (w3, b3) = mlp_params
    dh = z0.shape[1]
    return _pc(_mlp_body,
               out_shape=jax.ShapeDtypeStruct((z0.shape[0], w3.shape[1]), F32),
               )(z0, z1, w1[:dh], w1[dh:], b1[None, :], w2, b2[None, :],
                 w3, b3[None, :])


def kernel(x, params):
    h0, h1 = _feature_transform(x, params['ft_W'], params['ft_b'])
    outs0, outs1 = [h0], [h1]
    for l in range(4):
        lp = params['gat'][l]
        if l == 0:
            net0, net1 = outs0[0], outs1[0]
        else:
            net0 = jnp.concatenate(outs0[-2:], axis=1)
            net1 = jnp.concatenate(outs1[-2:], axis=1)
        gl0, gr0, nbr0 = _proj_and_knn(net0, lp['Wl'], lp['Wr'])
        gl1, gr1, nbr1 = _proj_and_knn(net1, lp['Wl'], lp['Wr'])
        gj0 = _gather_rows(gr0, nbr0.reshape(-1))
        gj1 = _gather_rows(gr1, nbr1.reshape(-1))
        outs0.append(_attention(gl0, gj0, lp['att'], lp['bias']))
        outs1.append(_attention(gl1, gj1, lp['att'], lp['bias']))
    return _mlp(outs0[-1], outs1[-1], params['mlp'])


# speedup vs baseline: 7.3434x; 1.0821x over previous
import functools

import jax
import jax.numpy as jnp
from jax import lax
from jax.experimental import pallas as pl
from jax.experimental.pallas import tpu as pltpu
from jax.experimental.pallas import tpu_sc as plsc

F32 = jnp.float32
HI = lax.Precision.DEFAULT

NBK = 2
KNN = 16
NV = 2048

_INTERPRET = False


def _pc(body, **kw):
    return pl.pallas_call(body, interpret=_INTERPRET, **kw)


def _ft_body(x_ref, w_ref, b_ref, o_ref):
    acc = jnp.dot(x_ref[...], w_ref[...], preferred_element_type=F32,
                  precision=HI)
    o_ref[...] = jnp.maximum(acc + b_ref[...], 0.0)


def _feature_transform(x, ft_W, ft_b):
    n, in_feat = x.shape
    tc = ft_W[0].shape[1]
    w6 = jnp.zeros((in_feat, NBK * tc), F32)
    w6 = w6.at[0:3, 0:tc].set(ft_W[0])
    w6 = w6.at[3:6, tc:2 * tc].set(ft_W[1])
    b6 = jnp.concatenate([ft_b[0], ft_b[1]])[None, :]
    out = _pc(_ft_body,
              out_shape=jax.ShapeDtypeStruct((n, NBK * tc), F32))(x, w6, b6)
    return out[:, :tc], out[:, tc:]


_TN = (((1,), (1,)), ((), ()))


def _pre_body(*refs, R, np_):
    hi_refs = refs[:np_]
    hf_refs = refs[np_:2 * np_]
    wl_refs = refs[2 * np_:3 * np_]
    wr_refs = refs[3 * np_:4 * np_]
    gl_ref, gr_ref, nbr_ref = refs[4 * np_:]
    r = pl.program_id(0)
    his = [h[...] for h in hi_refs]
    hfs = [h[...] for h in hf_refs]
    gl = sum(jnp.dot(hi, w[...], preferred_element_type=F32, precision=HI)
             for hi, w in zip(his, wl_refs))
    gr = sum(jnp.dot(hi, w[...], preferred_element_type=F32, precision=HI)
             for hi, w in zip(his, wr_refs))
    gl_ref[...] = gl
    gr_ref[...] = gr
    cross = sum(lax.dot_general(hi, hf, _TN, preferred_element_type=F32,
                                precision=HI)
                for hi, hf in zip(his, hfs))
    sqi = sum(jnp.sum(hi * hi, axis=1) for hi in his)
    sqj = sum(lax.dot_general(jnp.ones((8, hf.shape[1]), F32), hf * hf, _TN,
                              preferred_element_type=F32, precision=HI)
              for hf in hfs)[0:1]
    d = sqi[:, None] + sqj - 2.0 * cross
    rowid = lax.broadcasted_iota(jnp.int32, (R, NV), 0) + r * R
    colid = lax.broadcasted_iota(jnp.int32, (R, NV), 1)
    d = jnp.where(colid == rowid, jnp.float32(1e10), jnp.maximum(d, 0.0))
    key = (lax.bitcast_convert_type(d, jnp.int32) & ~jnp.int32(NV - 1)) | colid
    cols = []
    for _ in range(KNN):
        m = jnp.min(key, axis=1, keepdims=True)
        cols.append(m[:, 0] & jnp.int32(NV - 1))
        key = jnp.where(key == m, jnp.int32(0x7FFFFFFF), key)
    nbr_ref[...] = jnp.stack(cols, axis=1)


def _proj_and_knn(pieces, wl, wr, R=256):
    sizes = [p.shape[1] for p in pieces]
    np_ = len(pieces)
    dout = wl.shape[1]
    offs = [sum(sizes[:i]) for i in range(np_ + 1)]
    wl_parts = [wl[offs[i]:offs[i + 1]] for i in range(np_)]
    wr_parts = [wr[offs[i]:offs[i + 1]] for i in range(np_)]
    grid = (NV // R,)

    def spec_block(din_p):
        return pl.BlockSpec((R, din_p), lambda r: (r, 0))

    def spec_full(a, b):
        return pl.BlockSpec((a, b), lambda r: (0, 0))

    gl, gr, nbr = _pc(
        functools.partial(_pre_body, R=R, np_=np_),
        grid=grid,
        in_specs=([spec_block(s) for s in sizes]
                  + [spec_full(NV, s) for s in sizes]
                  + [spec_full(s, dout) for s in sizes]
                  + [spec_full(s, dout) for s in sizes]),
        out_specs=[
            pl.BlockSpec((R, dout), lambda r: (r, 0)),
            pl.BlockSpec((R, dout), lambda r: (r, 0)),
            pl.BlockSpec((R, KNN), lambda r: (r, 0)),
        ],
        out_shape=[
            jax.ShapeDtypeStruct((NV, dout), F32),
            jax.ShapeDtypeStruct((NV, dout), F32),
            jax.ShapeDtypeStruct((NV, KNN), jnp.int32),
        ],
    )(*pieces, *pieces, *wl_parts, *wr_parts)
    return gl, gr, nbr


def _att_body(gl_ref, gj_ref, attc_ref, bias_ref, o_ref, *, Bn, dout):
    gj = gj_ref[...]
    gl = gl_ref[...]
    glx = jnp.broadcast_to(gl[:, None, :], (Bn, KNN, dout)).reshape(
        Bn * KNN, dout)
    s = glx + gj
    lr = jnp.where(s > 0, s, 0.2 * s)
    e = jnp.dot(lr, attc_ref[...], preferred_element_type=F32,
                precision=HI)[:, 0]
    e1 = e.reshape(Bn, KNN)
    m = jnp.max(e1, axis=1, keepdims=True)
    ex = jnp.exp(e1 - m)
    alpha = ex / jnp.sum(ex, axis=1, keepdims=True)
    gj3 = gj.reshape(Bn, KNN, dout)
    out = jnp.sum(alpha[:, :, None] * gj3, axis=1) + bias_ref[...]
    o_ref[...] = jnp.maximum(out, 0.0)


def _attention(gl, gj_flat, att, bias, Bn=128):
    dout = gl.shape[1]
    rows = gl.shape[0]
    grid = (rows // Bn,)
    attc = jnp.broadcast_to(att[:, None], (dout, 128))
    return _pc(
        functools.partial(_att_body, Bn=Bn, dout=dout),
        grid=grid,
        in_specs=[
            pl.BlockSpec((Bn, dout), lambda i: (i, 0)),
            pl.BlockSpec((Bn * KNN, dout), lambda i: (i, 0)),
            pl.BlockSpec((dout, 128), lambda i: (0, 0)),
            pl.BlockSpec((1, dout), lambda i: (0, 0)),
        ],
        out_specs=pl.BlockSpec((Bn, dout), lambda i: (i, 0)),
        out_shape=jax.ShapeDtypeStruct((rows, dout), F32),
    )(gl, gj_flat, attc, bias[None, :])


def _gather_rows(table, idx):
    B = idx.shape[0]
    D = table.shape[1]
    NW = 32
    bpw = B // NW
    C = 64 if D >= 512 else 128
    nchunk = bpw // C
    mesh = plsc.VectorSubcoreMesh(core_axis_name="c", subcore_axis_name="s")

    @functools.partial(
        pl.kernel, mesh=mesh,
        out_type=jax.ShapeDtypeStruct((B, D), F32),
        scratch_types=[
            pltpu.VMEM((bpw,), jnp.int32),
            pltpu.VMEM((C, D), F32),
            pltpu.VMEM((C, D), F32),
            pltpu.SemaphoreType.DMA,
            pltpu.SemaphoreType.DMA,
        ],
    )
    def k(table_hbm, idx_hbm, out_hbm, idx_v, rows0_v, rows1_v, sem0, sem1):
        wid = lax.axis_index("s") * 2 + lax.axis_index("c")
        base = wid * bpw
        pltpu.sync_copy(idx_hbm.at[pl.ds(base, bpw)], idx_v)
        bufs = (rows0_v, rows1_v)
        sems = (sem0, sem1)
        pltpu.async_copy(table_hbm.at[idx_v.at[pl.ds(0, C)]], rows0_v, sem0)

        def body(p, carry):
            for bsel in range(2):
                g = p + bsel
                pltpu.make_async_copy(table_hbm.at[idx_v.at[pl.ds(0, C)]],
                                      bufs[bsel], sems[bsel]).wait()

                @pl.when(g + 1 < nchunk)
                def _():
                    pltpu.async_copy(
                        table_hbm.at[idx_v.at[pl.ds((g + 1) * C, C)]],
                        bufs[1 - bsel], sems[1 - bsel])

                pltpu.sync_copy(bufs[bsel],
                                out_hbm.at[pl.ds(base + g * C, C)])
            return carry

        lax.fori_loop(0, nchunk // 2, lambda p, c: body(p * 2, c), 0,
                      unroll=False)

    return k(table, idx)


def _mlp_body(z0_ref, z1_ref, w1a_ref, w1b_ref, b1_ref, w2_ref, b2_ref,
              w3_ref, b3_ref, o_ref):
    h = (jnp.dot(z0_ref[...], w1a_ref[...], preferred_element_type=F32,
                 precision=HI)
         + jnp.dot(z1_ref[...], w1b_ref[...], preferred_element_type=F32,
                   precision=HI) + b1_ref[...])
    h = jnp.maximum(h, 0.0)
    h = jnp.dot(h, w2_ref[...], preferred_element_type=F32,
                precision=HI) + b2_ref[...]
    h = jnp.maximum(h, 0.0)
    o_ref[...] = jnp.dot(h, w3_ref[...], preferred_element_type=F32,
                         precision=HI) + b3_ref[...]


def _mlp(z0, z1, mlp_params):
    (w1, b1), (w2, b2), (w3, b3) = mlp_params
    dh = z0.shape[1]
    return _pc(_mlp_body,
               out_shape=jax.ShapeDtypeStruct((z0.shape[0], w3.shape[1]), F32),
               )(z0, z1, w1[:dh], w1[dh:], b1[None, :], w2, b2[None, :],
                 w3, b3[None, :])


def kernel(x, params):
    h0, h1 = _feature_transform(x, params['ft_W'], params['ft_b'])
    outs0, outs1 = [h0], [h1]
    for l in range(4):
        lp = params['gat'][l]
        net0 = outs0[-2:] if l else [outs0[0]]
        net1 = outs1[-2:] if l else [outs1[0]]
        gl0, gr0, nbr0 = _proj_and_knn(net0, lp['Wl'], lp['Wr'])
        gl1, gr1, nbr1 = _proj_and_knn(net1, lp['Wl'], lp['Wr'])
        gj0 = _gather_rows(gr0, nbr0.reshape(-1))
        gj1 = _gather_rows(gr1, nbr1.reshape(-1))
        outs0.append(_attention(gl0, gj0, lp['att'], lp['bias']))
        outs1.append(_attention(gl1, gj1, lp['att'], lp['bias']))
    return _mlp(outs0[-1], outs1[-1], params['mlp'])
